# baseline (speedup 1.0000x reference)
import functools
import numpy as np
import jax
import jax.numpy as jnp
from jax import lax
from jax.experimental import pallas as pl
from jax.experimental.pallas import tpu as pltpu
from jax.experimental.pallas import tpu_sc as plsc

_B = 16
_ALPHA = 0.1
_EPS = 1e-8

_NC = 2
_NS = 16
_NW = _NC * _NS
_K = 200
_LG = 8


def _ema_weight_matrix():
    w = np.zeros((_B, _B), dtype=np.float32)
    w[0, 0] = 1.0
    for b in range(1, _B):
        w[b] = w[b - 1] * (1.0 - _ALPHA)
        w[b, b] = _ALPHA
    return w


_W = _ema_weight_matrix()


def _counts_kernel(ci, si, cnt_c, cnt_s, st_c, en_c, st_s, en_s):
    lane = lax.broadcasted_iota(jnp.int32, (1, _B), 1)
    acc_c = jnp.zeros((1, _B), jnp.float32)
    acc_s = jnp.zeros((1, _B), jnp.float32)
    acc_stc = jnp.zeros((1, _B), jnp.int32)
    acc_enc = jnp.zeros((1, _B), jnp.int32)
    acc_sts = jnp.zeros((1, _B), jnp.int32)
    acc_ens = jnp.zeros((1, _B), jnp.int32)
    xc = ci[...]
    xs = si[...]
    run_c = jnp.int32(0)
    run_s = jnp.int32(0)
    for s in range(_B):
        nc = jnp.sum((xc == s).astype(jnp.float32))
        ns = jnp.sum((xs == s).astype(jnp.float32))
        acc_c = acc_c + jnp.where(lane == s, nc, 0.0)
        acc_s = acc_s + jnp.where(lane == s, ns, 0.0)
        acc_stc = acc_stc + jnp.where(lane == s, run_c, 0)
        run_c = run_c + nc.astype(jnp.int32)
        acc_enc = acc_enc + jnp.where(lane == s, run_c, 0)
        acc_sts = acc_sts + jnp.where(lane == s, run_s, 0)
        run_s = run_s + ns.astype(jnp.int32)
        acc_ens = acc_ens + jnp.where(lane == s, run_s, 0)
    cnt_c[...] = acc_c
    cnt_s[...] = acc_s
    st_c[...] = acc_stc
    en_c[...] = acc_enc
    st_s[...] = acc_sts
    en_s[...] = acc_ens


def _cstats_kernel(r, base, cf, st, en, c_sum, c_ssq):
    gr = lax.broadcasted_iota(jnp.int32, (r, _B), 0) + (pl.program_id(0) * r + base)
    oh = ((gr >= st[...]) & (gr < en[...])).astype(jnp.float32)
    x = cf[...]
    dims = (((0,), (0,)), ((), ()))
    ps = lax.dot_general(oh, x, dims, preferred_element_type=jnp.float32)
    pq = lax.dot_general(oh, x * x, dims, preferred_element_type=jnp.float32)

    @pl.when(pl.program_id(0) == 0)
    def _():
        c_sum[...] = ps
        c_ssq[...] = pq

    @pl.when(pl.program_id(0) != 0)
    def _():
        c_sum[...] += ps
        c_ssq[...] += pq


def _sc_stats_kernel(n_sc, c, sf_hbm, cnt_s_hbm,
                     s_sum_hbm, s_ssq_hbm,
                     cnt_v, buf0, buf1, acc_sum, acc_ssq, sem0, sem1, sem_c):
    p = n_sc // _NW
    nchunk = p // _K
    wid = lax.axis_index("s") * _NC + lax.axis_index("c")
    lo = wid * p

    def one_array(feat_hbm, cnt_hbm, sum_hbm, ssq_hbm):
        pltpu.make_async_copy(cnt_hbm, cnt_v, sem_c).start()
        pltpu.make_async_copy(feat_hbm.at[pl.ds(lo, _K)], buf0, sem0).start()
        pltpu.make_async_copy(feat_hbm.at[pl.ds(lo + _K, _K)], buf1, sem1).start()
        pltpu.make_async_copy(cnt_hbm, cnt_v, sem_c).wait()

        cv = cnt_v[...]
        bnd = [jnp.int32(0)]
        run = jnp.int32(0)
        for s in range(_B):
            run = run + cv[s].astype(jnp.int32)
            bnd.append(jnp.minimum(jnp.maximum(run - lo, 0), p))

        for s in range(_B):
            for g in range(_LG):
                acc_sum[s, pl.ds(g * 16, 16)] = jnp.zeros((16,), jnp.float32)
                acc_ssq[s, pl.ds(g * 16, 16)] = jnp.zeros((16,), jnp.float32)

        def process(buf, cb):
            for s in range(_B):
                r0 = jnp.maximum(bnd[s] - cb, 0)
                r1 = jnp.minimum(bnd[s + 1] - cb, _K)

                @pl.when(r1 > r0)
                def _():
                    sums = tuple(acc_sum[s, pl.ds(g * 16, 16)] for g in range(_LG))
                    ssqs = tuple(acc_ssq[s, pl.ds(g * 16, 16)] for g in range(_LG))

                    def row_body(r, carry):
                        su, sq = carry
                        nsu = []
                        nsq = []
                        for g in range(_LG):
                            x = buf[r, pl.ds(g * 16, 16)]
                            nsu.append(su[g] + x)
                            nsq.append(sq[g] + x * x)
                        return (tuple(nsu), tuple(nsq))

                    sums, ssqs = lax.fori_loop(r0, r1, row_body, (sums, ssqs))
                    for g in range(_LG):
                        acc_sum[s, pl.ds(g * 16, 16)] = sums[g]
                        acc_ssq[s, pl.ds(g * 16, 16)] = ssqs[g]

        def chunk_body(t, carry):
            j0 = 2 * t
            cb0 = j0 * _K
            pltpu.make_async_copy(feat_hbm.at[pl.ds(lo + cb0, _K)], buf0, sem0).wait()
            process(buf0, cb0)

            @pl.when(j0 + 2 < nchunk)
            def _():
                pltpu.make_async_copy(
                    feat_hbm.at[pl.ds(lo + (j0 + 2) * _K, _K)], buf0, sem0).start()

            cb1 = (j0 + 1) * _K
            pltpu.make_async_copy(feat_hbm.at[pl.ds(lo + cb1, _K)], buf1, sem1).wait()
            process(buf1, cb1)

            @pl.when(j0 + 3 < nchunk)
            def _():
                pltpu.make_async_copy(
                    feat_hbm.at[pl.ds(lo + (j0 + 3) * _K, _K)], buf1, sem1).start()

            return carry

        lax.fori_loop(0, nchunk // 2, chunk_body, 0)

        pltpu.make_async_copy(acc_sum, sum_hbm.at[wid], sem0).start()
        pltpu.make_async_copy(acc_sum, sum_hbm.at[wid], sem0).wait()
        pltpu.make_async_copy(acc_ssq, ssq_hbm.at[wid], sem1).start()
        pltpu.make_async_copy(acc_ssq, ssq_hbm.at[wid], sem1).wait()

    one_array(sf_hbm, cnt_s_hbm, s_sum_hbm, s_ssq_hbm)


def _finalize_kernel(c_sum, c_ssq, s_sum, s_ssq, s_sum_t, s_ssq_t,
                     cnt_c, cnt_s, w, scale, offset):
    csum = c_sum[...]
    cssq = c_ssq[...]
    ssum = jnp.sum(s_sum[...], axis=0) + s_sum_t[...]
    sssq = jnp.sum(s_ssq[...], axis=0) + s_ssq_t[...]
    cw = csum.shape[1]
    dims = (((1,), (0,)), ((), ()))
    lane = lax.broadcasted_iota(jnp.int32, (1, _B), 1)
    rowi = lax.broadcasted_iota(jnp.int32, (_B, cw), 0)
    ccnt = jnp.zeros((_B, cw), jnp.float32)
    scnt = jnp.zeros((_B, cw), jnp.float32)
    for s in range(_B):
        nc = jnp.sum(jnp.where(lane == s, cnt_c[...], 0.0))
        nss = jnp.sum(jnp.where(lane == s, cnt_s[...], 0.0))
        ccnt = jnp.where(rowi == s, nc, ccnt)
        scnt = jnp.where(rowi == s, nss, scnt)
    cmean = csum / ccnt
    cvar = (cssq - ccnt * cmean * cmean) / (ccnt - 1.0)
    cstd = jnp.sqrt(jnp.maximum(cvar, 0.0)) + _EPS
    smean = ssum / scnt
    svar = (sssq - scnt * smean * smean) / (scnt - 1.0)
    sstd = jnp.sqrt(jnp.maximum(svar, 0.0)) + _EPS
    gmean = lax.dot_general(w[...], smean, dims, preferred_element_type=jnp.float32)
    gstd = lax.dot_general(w[...], sstd, dims, preferred_element_type=jnp.float32)
    sc = gstd / cstd
    scale[...] = sc
    offset[...] = gmean - sc * cmean


def _norm_kernel(r, cf, st, en, scale, offset, out):
    gr = lax.broadcasted_iota(jnp.int32, (r, _B), 0) + pl.program_id(0) * r
    oh = ((gr >= st[...]) & (gr < en[...])).astype(jnp.float32)
    dims = (((1,), (0,)), ((), ()))
    rs = lax.dot_general(oh, scale[...], dims, preferred_element_type=jnp.float32)
    ro = lax.dot_general(oh, offset[...], dims, preferred_element_type=jnp.float32)
    out[...] = cf[...] * rs + ro


def _pick_block(n):
    for r in (8000, 4000, 2000, 1600, 1000, 800, 500, 400, 200, 100, 8):
        if n % r == 0:
            return r
    return None


def kernel(content_feats, style_feats, content_batch_indices, style_batch_indices):
    n, c = content_feats.shape
    ns = style_feats.shape[0]
    assert ns == n, "kernel assumes matching content/style row counts"
    assert n % (_NW * _K) == 0 and c == 128 and n % 128 == 0

    ci2 = content_batch_indices.reshape(n // 128, 128)
    si2 = style_batch_indices.reshape(n // 128, 128)
    cnt_shape = jax.ShapeDtypeStruct((1, _B), jnp.float32)
    bnd_shape = jax.ShapeDtypeStruct((1, _B), jnp.int32)
    cnt_c, cnt_s, st_c, en_c, st_s, en_s = pl.pallas_call(
        _counts_kernel,
        out_shape=[cnt_shape, cnt_shape, bnd_shape, bnd_shape,
                   bnd_shape, bnd_shape],
    )(ci2, si2)

    x_sc = (int(n * 0.72) // (_NW * _K)) * (_NW * _K)
    part = jax.ShapeDtypeStruct((_NW, _B, c), jnp.float32)
    mesh = plsc.VectorSubcoreMesh(core_axis_name="c", subcore_axis_name="s")
    sc_stats = pl.kernel(
        functools.partial(_sc_stats_kernel, x_sc, c),
        mesh=mesh,
        out_type=[part] * 2,
        scratch_types=[
            pltpu.VMEM((_B,), jnp.float32),
            pltpu.VMEM((_K, c), jnp.float32),
            pltpu.VMEM((_K, c), jnp.float32),
            pltpu.VMEM((_B, c), jnp.float32),
            pltpu.VMEM((_B, c), jnp.float32),
            pltpu.SemaphoreType.DMA,
            pltpu.SemaphoreType.DMA,
            pltpu.SemaphoreType.DMA,
        ],
    )
    s_sum, s_ssq = sc_stats(style_feats, cnt_s.reshape(_B))

    rb = _pick_block(n)
    nbb = n // rb
    stat_shape = jax.ShapeDtypeStruct((_B, c), jnp.float32)
    stat_spec = pl.BlockSpec((_B, c), lambda i: (0, 0))
    bspec = pl.BlockSpec((1, _B), lambda i: (0, 0))
    c_sum, c_ssq = pl.pallas_call(
        functools.partial(_cstats_kernel, rb, 0),
        grid=(nbb,),
        in_specs=[
            pl.BlockSpec((rb, c), lambda i: (i, 0)),
            bspec,
            bspec,
        ],
        out_specs=[stat_spec, stat_spec],
        out_shape=[stat_shape, stat_shape],
    )(content_feats, st_c, en_c)

    tail = n - x_sc
    rb_t = None
    for cand in (6400, 3200, 1600, 800, 400, 200, 100, 8):
        if tail % cand == 0:
            rb_t = cand
            break
    off_t = x_sc // rb_t
    s_sum_t, s_ssq_t = pl.pallas_call(
        functools.partial(_cstats_kernel, rb_t, x_sc),
        grid=(tail // rb_t,),
        in_specs=[
            pl.BlockSpec((rb_t, c), lambda i: (i + off_t, 0)),
            bspec,
            bspec,
        ],
        out_specs=[stat_spec, stat_spec],
        out_shape=[stat_shape, stat_shape],
    )(style_feats, st_s, en_s)

    w = jnp.asarray(_W)
    scale, offset = pl.pallas_call(
        _finalize_kernel,
        out_shape=[stat_shape, stat_shape],
    )(c_sum, c_ssq, s_sum, s_ssq, s_sum_t, s_ssq_t, cnt_c, cnt_s, w)

    r = _pick_block(n)
    nb = n // r
    bnd_spec = pl.BlockSpec((1, _B), lambda i: (0, 0))
    out = pl.pallas_call(
        functools.partial(_norm_kernel, r),
        grid=(nb,),
        in_specs=[
            pl.BlockSpec((r, c), lambda i: (i, 0)),
            bnd_spec,
            bnd_spec,
            pl.BlockSpec((_B, c), lambda i: (0, 0)),
            pl.BlockSpec((_B, c), lambda i: (0, 0)),
        ],
        out_specs=pl.BlockSpec((r, c), lambda i: (i, 0)),
        out_shape=jax.ShapeDtypeStruct((n, c), jnp.float32),
        compiler_params=pltpu.CompilerParams(
            dimension_semantics=("parallel",)),
    )(content_feats, st_c, en_c, scale, offset)
    return out


# speedup vs baseline: 1.0511x; 1.0511x over previous
import functools
import numpy as np
import jax
import jax.numpy as jnp
from jax import lax
from jax.experimental import pallas as pl
from jax.experimental.pallas import tpu as pltpu
from jax.experimental.pallas import tpu_sc as plsc

_B = 16
_ALPHA = 0.1
_EPS = 1e-8

_NC = 2
_NS = 16
_NW = _NC * _NS
_K = 200
_LG = 8


def _ema_weight_matrix():
    w = np.zeros((_B, _B), dtype=np.float32)
    w[0, 0] = 1.0
    for b in range(1, _B):
        w[b] = w[b - 1] * (1.0 - _ALPHA)
        w[b, b] = _ALPHA
    return w


_W = _ema_weight_matrix()


def _counts_kernel(ci, si, cnt_c, cnt_s, st_c, en_c, st_s, en_s):
    lane = lax.broadcasted_iota(jnp.int32, (1, _B), 1)
    acc_c = jnp.zeros((1, _B), jnp.float32)
    acc_s = jnp.zeros((1, _B), jnp.float32)
    acc_stc = jnp.zeros((1, _B), jnp.int32)
    acc_enc = jnp.zeros((1, _B), jnp.int32)
    acc_sts = jnp.zeros((1, _B), jnp.int32)
    acc_ens = jnp.zeros((1, _B), jnp.int32)
    xc = ci[...]
    xs = si[...]
    run_c = jnp.int32(0)
    run_s = jnp.int32(0)
    for s in range(_B):
        nc = jnp.sum((xc == s).astype(jnp.float32))
        ns = jnp.sum((xs == s).astype(jnp.float32))
        acc_c = acc_c + jnp.where(lane == s, nc, 0.0)
        acc_s = acc_s + jnp.where(lane == s, ns, 0.0)
        acc_stc = acc_stc + jnp.where(lane == s, run_c, 0)
        run_c = run_c + nc.astype(jnp.int32)
        acc_enc = acc_enc + jnp.where(lane == s, run_c, 0)
        acc_sts = acc_sts + jnp.where(lane == s, run_s, 0)
        run_s = run_s + ns.astype(jnp.int32)
        acc_ens = acc_ens + jnp.where(lane == s, run_s, 0)
    cnt_c[...] = acc_c
    cnt_s[...] = acc_s
    st_c[...] = acc_stc
    en_c[...] = acc_enc
    st_s[...] = acc_sts
    en_s[...] = acc_ens


def _cstats_kernel(r, base, cf, st, en, c_sum, c_ssq):
    gr = lax.broadcasted_iota(jnp.int32, (r, _B), 0) + (pl.program_id(0) * r + base)
    oh = ((gr >= st[...]) & (gr < en[...])).astype(jnp.float32)
    x = cf[...]
    dims = (((0,), (0,)), ((), ()))
    ps = lax.dot_general(oh, x, dims, preferred_element_type=jnp.float32)
    pq = lax.dot_general(oh, x * x, dims, preferred_element_type=jnp.float32)

    @pl.when(pl.program_id(0) == 0)
    def _():
        c_sum[...] = ps
        c_ssq[...] = pq

    @pl.when(pl.program_id(0) != 0)
    def _():
        c_sum[...] += ps
        c_ssq[...] += pq


def _sc_stats_kernel(n_sc, c, sf_hbm, cnt_s_hbm,
                     s_sum_hbm, s_ssq_hbm,
                     cnt_v, buf0, buf1, acc_sum, acc_ssq, sem0, sem1, sem_c):
    p = n_sc // _NW
    nchunk = p // _K
    wid = lax.axis_index("s") * _NC + lax.axis_index("c")
    lo = wid * p

    def one_array(feat_hbm, cnt_hbm, sum_hbm, ssq_hbm):
        pltpu.make_async_copy(cnt_hbm, cnt_v, sem_c).start()
        pltpu.make_async_copy(feat_hbm.at[pl.ds(lo, _K)], buf0, sem0).start()
        pltpu.make_async_copy(feat_hbm.at[pl.ds(lo + _K, _K)], buf1, sem1).start()
        pltpu.make_async_copy(cnt_hbm, cnt_v, sem_c).wait()

        cv = cnt_v[...]
        bnd = [jnp.int32(0)]
        run = jnp.int32(0)
        for s in range(_B):
            run = run + cv[s].astype(jnp.int32)
            bnd.append(jnp.minimum(jnp.maximum(run - lo, 0), p))

        for s in range(_B):
            for g in range(_LG):
                acc_sum[s, pl.ds(g * 16, 16)] = jnp.zeros((16,), jnp.float32)
                acc_ssq[s, pl.ds(g * 16, 16)] = jnp.zeros((16,), jnp.float32)

        def process(buf, cb):
            for s in range(_B):
                r0 = jnp.maximum(bnd[s] - cb, 0)
                r1 = jnp.minimum(bnd[s + 1] - cb, _K)

                @pl.when(r1 > r0)
                def _():
                    sums = tuple(acc_sum[s, pl.ds(g * 16, 16)] for g in range(_LG))
                    ssqs = tuple(acc_ssq[s, pl.ds(g * 16, 16)] for g in range(_LG))

                    def row_body(r, carry):
                        su, sq = carry
                        nsu = []
                        nsq = []
                        for g in range(_LG):
                            x = buf[r, pl.ds(g * 16, 16)]
                            nsu.append(su[g] + x)
                            nsq.append(sq[g] + x * x)
                        return (tuple(nsu), tuple(nsq))

                    sums, ssqs = lax.fori_loop(r0, r1, row_body, (sums, ssqs))
                    for g in range(_LG):
                        acc_sum[s, pl.ds(g * 16, 16)] = sums[g]
                        acc_ssq[s, pl.ds(g * 16, 16)] = ssqs[g]

        def chunk_body(t, carry):
            j0 = 2 * t
            cb0 = j0 * _K
            pltpu.make_async_copy(feat_hbm.at[pl.ds(lo + cb0, _K)], buf0, sem0).wait()
            process(buf0, cb0)

            @pl.when(j0 + 2 < nchunk)
            def _():
                pltpu.make_async_copy(
                    feat_hbm.at[pl.ds(lo + (j0 + 2) * _K, _K)], buf0, sem0).start()

            cb1 = (j0 + 1) * _K
            pltpu.make_async_copy(feat_hbm.at[pl.ds(lo + cb1, _K)], buf1, sem1).wait()
            process(buf1, cb1)

            @pl.when(j0 + 3 < nchunk)
            def _():
                pltpu.make_async_copy(
                    feat_hbm.at[pl.ds(lo + (j0 + 3) * _K, _K)], buf1, sem1).start()

            return carry

        lax.fori_loop(0, nchunk // 2, chunk_body, 0)

        pltpu.make_async_copy(acc_sum, sum_hbm.at[wid], sem0).start()
        pltpu.make_async_copy(acc_sum, sum_hbm.at[wid], sem0).wait()
        pltpu.make_async_copy(acc_ssq, ssq_hbm.at[wid], sem1).start()
        pltpu.make_async_copy(acc_ssq, ssq_hbm.at[wid], sem1).wait()

    one_array(sf_hbm, cnt_s_hbm, s_sum_hbm, s_ssq_hbm)


def _finalize_kernel(c_sum, c_ssq, s_sum, s_ssq, s_sum_t, s_ssq_t,
                     cnt_c, cnt_s, w, scale, offset):
    csum = c_sum[...]
    cssq = c_ssq[...]
    ssum = jnp.sum(s_sum[...], axis=0) + s_sum_t[...]
    sssq = jnp.sum(s_ssq[...], axis=0) + s_ssq_t[...]
    cw = csum.shape[1]
    dims = (((1,), (0,)), ((), ()))
    lane = lax.broadcasted_iota(jnp.int32, (1, _B), 1)
    rowi = lax.broadcasted_iota(jnp.int32, (_B, cw), 0)
    ccnt = jnp.zeros((_B, cw), jnp.float32)
    scnt = jnp.zeros((_B, cw), jnp.float32)
    for s in range(_B):
        nc = jnp.sum(jnp.where(lane == s, cnt_c[...], 0.0))
        nss = jnp.sum(jnp.where(lane == s, cnt_s[...], 0.0))
        ccnt = jnp.where(rowi == s, nc, ccnt)
        scnt = jnp.where(rowi == s, nss, scnt)
    cmean = csum / ccnt
    cvar = (cssq - ccnt * cmean * cmean) / (ccnt - 1.0)
    cstd = jnp.sqrt(jnp.maximum(cvar, 0.0)) + _EPS
    smean = ssum / scnt
    svar = (sssq - scnt * smean * smean) / (scnt - 1.0)
    sstd = jnp.sqrt(jnp.maximum(svar, 0.0)) + _EPS
    gmean = lax.dot_general(w[...], smean, dims, preferred_element_type=jnp.float32)
    gstd = lax.dot_general(w[...], sstd, dims, preferred_element_type=jnp.float32)
    sc = gstd / cstd
    scale[...] = sc
    offset[...] = gmean - sc * cmean


def _norm_kernel(r, cf, st, en, scale, offset, out):
    gr = lax.broadcasted_iota(jnp.int32, (r, _B), 0) + pl.program_id(0) * r
    oh = ((gr >= st[...]) & (gr < en[...])).astype(jnp.float32)
    dims = (((1,), (0,)), ((), ()))
    rs = lax.dot_general(oh, scale[...], dims, preferred_element_type=jnp.float32)
    ro = lax.dot_general(oh, offset[...], dims, preferred_element_type=jnp.float32)
    out[...] = cf[...] * rs + ro


def _pick_block(n):
    for r in (16000, 8000, 4000, 2000, 1600, 1000, 800, 500, 400, 200, 100, 8):
        if n % r == 0:
            return r
    return None


def kernel(content_feats, style_feats, content_batch_indices, style_batch_indices):
    n, c = content_feats.shape
    ns = style_feats.shape[0]
    assert ns == n, "kernel assumes matching content/style row counts"
    assert n % (_NW * _K) == 0 and c == 128 and n % 128 == 0

    ci2 = content_batch_indices.reshape(n // 128, 128)
    si2 = style_batch_indices.reshape(n // 128, 128)
    cnt_shape = jax.ShapeDtypeStruct((1, _B), jnp.float32)
    bnd_shape = jax.ShapeDtypeStruct((1, _B), jnp.int32)
    cnt_c, cnt_s, st_c, en_c, st_s, en_s = pl.pallas_call(
        _counts_kernel,
        out_shape=[cnt_shape, cnt_shape, bnd_shape, bnd_shape,
                   bnd_shape, bnd_shape],
    )(ci2, si2)

    x_sc = (int(n * 0.80) // (_NW * _K)) * (_NW * _K)
    part = jax.ShapeDtypeStruct((_NW, _B, c), jnp.float32)
    mesh = plsc.VectorSubcoreMesh(core_axis_name="c", subcore_axis_name="s")
    sc_stats = pl.kernel(
        functools.partial(_sc_stats_kernel, x_sc, c),
        mesh=mesh,
        out_type=[part] * 2,
        scratch_types=[
            pltpu.VMEM((_B,), jnp.float32),
            pltpu.VMEM((_K, c), jnp.float32),
            pltpu.VMEM((_K, c), jnp.float32),
            pltpu.VMEM((_B, c), jnp.float32),
            pltpu.VMEM((_B, c), jnp.float32),
            pltpu.SemaphoreType.DMA,
            pltpu.SemaphoreType.DMA,
            pltpu.SemaphoreType.DMA,
        ],
    )
    s_sum, s_ssq = sc_stats(style_feats, cnt_s.reshape(_B))

    rb = _pick_block(n)
    nbb = n // rb
    stat_shape = jax.ShapeDtypeStruct((_B, c), jnp.float32)
    stat_spec = pl.BlockSpec((_B, c), lambda i: (0, 0))
    bspec = pl.BlockSpec((1, _B), lambda i: (0, 0))
    c_sum, c_ssq = pl.pallas_call(
        functools.partial(_cstats_kernel, rb, 0),
        grid=(nbb,),
        in_specs=[
            pl.BlockSpec((rb, c), lambda i: (i, 0)),
            bspec,
            bspec,
        ],
        out_specs=[stat_spec, stat_spec],
        out_shape=[stat_shape, stat_shape],
    )(content_feats, st_c, en_c)

    tail = n - x_sc
    rb_t = None
    for cand in (6400, 3200, 1600, 800, 400, 200, 100, 8):
        if tail % cand == 0:
            rb_t = cand
            break
    off_t = x_sc // rb_t
    s_sum_t, s_ssq_t = pl.pallas_call(
        functools.partial(_cstats_kernel, rb_t, x_sc),
        grid=(tail // rb_t,),
        in_specs=[
            pl.BlockSpec((rb_t, c), lambda i: (i + off_t, 0)),
            bspec,
            bspec,
        ],
        out_specs=[stat_spec, stat_spec],
        out_shape=[stat_shape, stat_shape],
    )(style_feats, st_s, en_s)

    w = jnp.asarray(_W)
    scale, offset = pl.pallas_call(
        _finalize_kernel,
        out_shape=[stat_shape, stat_shape],
    )(c_sum, c_ssq, s_sum, s_ssq, s_sum_t, s_ssq_t, cnt_c, cnt_s, w)

    r = _pick_block(n)
    nb = n // r
    bnd_spec = pl.BlockSpec((1, _B), lambda i: (0, 0))
    out = pl.pallas_call(
        functools.partial(_norm_kernel, r),
        grid=(nb,),
        in_specs=[
            pl.BlockSpec((r, c), lambda i: (i, 0)),
            bnd_spec,
            bnd_spec,
            pl.BlockSpec((_B, c), lambda i: (0, 0)),
            pl.BlockSpec((_B, c), lambda i: (0, 0)),
        ],
        out_specs=pl.BlockSpec((r, c), lambda i: (i, 0)),
        out_shape=jax.ShapeDtypeStruct((n, c), jnp.float32),
        compiler_params=pltpu.CompilerParams(
            dimension_semantics=("parallel",)),
    )(content_feats, st_c, en_c, scale, offset)
    return out


# speedup vs baseline: 1.0552x; 1.0039x over previous
import functools
import numpy as np
import jax
import jax.numpy as jnp
from jax import lax
from jax.experimental import pallas as pl
from jax.experimental.pallas import tpu as pltpu
from jax.experimental.pallas import tpu_sc as plsc

_B = 16
_ALPHA = 0.1
_EPS = 1e-8

_NC = 2
_NS = 16
_NW = _NC * _NS
_K = 200
_LG = 8


def _ema_weight_matrix():
    w = np.zeros((_B, _B), dtype=np.float32)
    w[0, 0] = 1.0
    for b in range(1, _B):
        w[b] = w[b - 1] * (1.0 - _ALPHA)
        w[b, b] = _ALPHA
    return w


_W = _ema_weight_matrix()


def _counts_kernel(ci, si, cnt_c, cnt_s, st_c, en_c, st_s, en_s):
    lane = lax.broadcasted_iota(jnp.int32, (1, _B), 1)
    acc_c = jnp.zeros((1, _B), jnp.float32)
    acc_s = jnp.zeros((1, _B), jnp.float32)
    acc_stc = jnp.zeros((1, _B), jnp.int32)
    acc_enc = jnp.zeros((1, _B), jnp.int32)
    acc_sts = jnp.zeros((1, _B), jnp.int32)
    acc_ens = jnp.zeros((1, _B), jnp.int32)
    xc = ci[...]
    xs = si[...]
    run_c = jnp.int32(0)
    run_s = jnp.int32(0)
    for s in range(_B):
        nc = jnp.sum((xc == s).astype(jnp.float32))
        ns = jnp.sum((xs == s).astype(jnp.float32))
        acc_c = acc_c + jnp.where(lane == s, nc, 0.0)
        acc_s = acc_s + jnp.where(lane == s, ns, 0.0)
        acc_stc = acc_stc + jnp.where(lane == s, run_c, 0)
        run_c = run_c + nc.astype(jnp.int32)
        acc_enc = acc_enc + jnp.where(lane == s, run_c, 0)
        acc_sts = acc_sts + jnp.where(lane == s, run_s, 0)
        run_s = run_s + ns.astype(jnp.int32)
        acc_ens = acc_ens + jnp.where(lane == s, run_s, 0)
    cnt_c[...] = acc_c
    cnt_s[...] = acc_s
    st_c[...] = acc_stc
    en_c[...] = acc_enc
    st_s[...] = acc_sts
    en_s[...] = acc_ens


def _cstats_kernel(r, base, cf, st, en, c_sum, c_ssq):
    gr = lax.broadcasted_iota(jnp.int32, (r, _B), 0) + (pl.program_id(0) * r + base)
    oh = ((gr >= st[...]) & (gr < en[...])).astype(jnp.float32)
    x = cf[...]
    dims = (((0,), (0,)), ((), ()))
    ps = lax.dot_general(oh, x, dims, preferred_element_type=jnp.float32)
    pq = lax.dot_general(oh, x * x, dims, preferred_element_type=jnp.float32)

    @pl.when(pl.program_id(0) == 0)
    def _():
        c_sum[...] = ps
        c_ssq[...] = pq

    @pl.when(pl.program_id(0) != 0)
    def _():
        c_sum[...] += ps
        c_ssq[...] += pq


def _sc_stats_kernel(n_sc, c, sf_hbm, cnt_s_hbm,
                     s_sum_hbm, s_ssq_hbm,
                     cnt_v, buf0, buf1, acc_sum, acc_ssq, sem0, sem1, sem_c):
    p = n_sc // _NW
    nchunk = p // _K
    wid = lax.axis_index("s") * _NC + lax.axis_index("c")
    lo = wid * p

    def one_array(feat_hbm, cnt_hbm, sum_hbm, ssq_hbm):
        pltpu.make_async_copy(cnt_hbm, cnt_v, sem_c).start()
        pltpu.make_async_copy(feat_hbm.at[pl.ds(lo, _K)], buf0, sem0).start()
        pltpu.make_async_copy(feat_hbm.at[pl.ds(lo + _K, _K)], buf1, sem1).start()
        pltpu.make_async_copy(cnt_hbm, cnt_v, sem_c).wait()

        cv = cnt_v[...]
        bnd = [jnp.int32(0)]
        run = jnp.int32(0)
        for s in range(_B):
            run = run + cv[s].astype(jnp.int32)
            bnd.append(jnp.minimum(jnp.maximum(run - lo, 0), p))

        for s in range(_B):
            for g in range(_LG):
                acc_sum[s, pl.ds(g * 16, 16)] = jnp.zeros((16,), jnp.float32)
                acc_ssq[s, pl.ds(g * 16, 16)] = jnp.zeros((16,), jnp.float32)

        def process(buf, cb):
            for s in range(_B):
                r0 = jnp.maximum(bnd[s] - cb, 0)
                r1 = jnp.minimum(bnd[s + 1] - cb, _K)

                @pl.when(r1 > r0)
                def _():
                    sums = tuple(acc_sum[s, pl.ds(g * 16, 16)] for g in range(_LG))
                    ssqs = tuple(acc_ssq[s, pl.ds(g * 16, 16)] for g in range(_LG))

                    def row_body(r, carry):
                        su, sq = carry
                        nsu = []
                        nsq = []
                        for g in range(_LG):
                            x = buf[r, pl.ds(g * 16, 16)]
                            nsu.append(su[g] + x)
                            nsq.append(sq[g] + x * x)
                        return (tuple(nsu), tuple(nsq))

                    sums, ssqs = lax.fori_loop(r0, r1, row_body, (sums, ssqs))
                    for g in range(_LG):
                        acc_sum[s, pl.ds(g * 16, 16)] = sums[g]
                        acc_ssq[s, pl.ds(g * 16, 16)] = ssqs[g]

        def chunk_body(t, carry):
            j0 = 2 * t
            cb0 = j0 * _K
            pltpu.make_async_copy(feat_hbm.at[pl.ds(lo + cb0, _K)], buf0, sem0).wait()
            process(buf0, cb0)

            @pl.when(j0 + 2 < nchunk)
            def _():
                pltpu.make_async_copy(
                    feat_hbm.at[pl.ds(lo + (j0 + 2) * _K, _K)], buf0, sem0).start()

            cb1 = (j0 + 1) * _K
            pltpu.make_async_copy(feat_hbm.at[pl.ds(lo + cb1, _K)], buf1, sem1).wait()
            process(buf1, cb1)

            @pl.when(j0 + 3 < nchunk)
            def _():
                pltpu.make_async_copy(
                    feat_hbm.at[pl.ds(lo + (j0 + 3) * _K, _K)], buf1, sem1).start()

            return carry

        lax.fori_loop(0, nchunk // 2, chunk_body, 0)

        pltpu.make_async_copy(acc_sum, sum_hbm.at[wid], sem0).start()
        pltpu.make_async_copy(acc_sum, sum_hbm.at[wid], sem0).wait()
        pltpu.make_async_copy(acc_ssq, ssq_hbm.at[wid], sem1).start()
        pltpu.make_async_copy(acc_ssq, ssq_hbm.at[wid], sem1).wait()

    one_array(sf_hbm, cnt_s_hbm, s_sum_hbm, s_ssq_hbm)


def _finalize_kernel(c_sum, c_ssq, s_sum, s_ssq, s_sum_t, s_ssq_t,
                     cnt_c, cnt_s, w, scale, offset):
    csum = c_sum[...]
    cssq = c_ssq[...]
    ssum = jnp.sum(s_sum[...], axis=0) + s_sum_t[...]
    sssq = jnp.sum(s_ssq[...], axis=0) + s_ssq_t[...]
    cw = csum.shape[1]
    dims = (((1,), (0,)), ((), ()))
    lane = lax.broadcasted_iota(jnp.int32, (1, _B), 1)
    rowi = lax.broadcasted_iota(jnp.int32, (_B, cw), 0)
    ccnt = jnp.zeros((_B, cw), jnp.float32)
    scnt = jnp.zeros((_B, cw), jnp.float32)
    for s in range(_B):
        nc = jnp.sum(jnp.where(lane == s, cnt_c[...], 0.0))
        nss = jnp.sum(jnp.where(lane == s, cnt_s[...], 0.0))
        ccnt = jnp.where(rowi == s, nc, ccnt)
        scnt = jnp.where(rowi == s, nss, scnt)
    cmean = csum / ccnt
    cvar = (cssq - ccnt * cmean * cmean) / (ccnt - 1.0)
    cstd = jnp.sqrt(jnp.maximum(cvar, 0.0)) + _EPS
    smean = ssum / scnt
    svar = (sssq - scnt * smean * smean) / (scnt - 1.0)
    sstd = jnp.sqrt(jnp.maximum(svar, 0.0)) + _EPS
    gmean = lax.dot_general(w[...], smean, dims, preferred_element_type=jnp.float32)
    gstd = lax.dot_general(w[...], sstd, dims, preferred_element_type=jnp.float32)
    sc = gstd / cstd
    scale[...] = sc
    offset[...] = gmean - sc * cmean


def _norm_kernel(r, cf, st, en, scale, offset, out):
    gr = lax.broadcasted_iota(jnp.int32, (r, _B), 0) + pl.program_id(0) * r
    oh = ((gr >= st[...]) & (gr < en[...])).astype(jnp.float32)
    dims = (((1,), (0,)), ((), ()))
    rs = lax.dot_general(oh, scale[...], dims, preferred_element_type=jnp.float32)
    ro = lax.dot_general(oh, offset[...], dims, preferred_element_type=jnp.float32)
    out[...] = cf[...] * rs + ro


def _pick_block(n):
    for r in (16000, 8000, 4000, 2000, 1600, 1000, 800, 500, 400, 200, 100, 8):
        if n % r == 0:
            return r
    return None


def kernel(content_feats, style_feats, content_batch_indices, style_batch_indices):
    n, c = content_feats.shape
    ns = style_feats.shape[0]
    assert ns == n, "kernel assumes matching content/style row counts"
    assert n % (_NW * _K) == 0 and c == 128 and n % 128 == 0

    ci2 = content_batch_indices.reshape(n // 128, 128)
    si2 = style_batch_indices.reshape(n // 128, 128)
    cnt_shape = jax.ShapeDtypeStruct((1, _B), jnp.float32)
    bnd_shape = jax.ShapeDtypeStruct((1, _B), jnp.int32)
    cnt_c, cnt_s, st_c, en_c, st_s, en_s = pl.pallas_call(
        _counts_kernel,
        out_shape=[cnt_shape, cnt_shape, bnd_shape, bnd_shape,
                   bnd_shape, bnd_shape],
    )(ci2, si2)

    x_sc = (int(n * 0.84) // (_NW * _K)) * (_NW * _K)
    part = jax.ShapeDtypeStruct((_NW, _B, c), jnp.float32)
    mesh = plsc.VectorSubcoreMesh(core_axis_name="c", subcore_axis_name="s")
    sc_stats = pl.kernel(
        functools.partial(_sc_stats_kernel, x_sc, c),
        mesh=mesh,
        out_type=[part] * 2,
        scratch_types=[
            pltpu.VMEM((_B,), jnp.float32),
            pltpu.VMEM((_K, c), jnp.float32),
            pltpu.VMEM((_K, c), jnp.float32),
            pltpu.VMEM((_B, c), jnp.float32),
            pltpu.VMEM((_B, c), jnp.float32),
            pltpu.SemaphoreType.DMA,
            pltpu.SemaphoreType.DMA,
            pltpu.SemaphoreType.DMA,
        ],
    )
    s_sum, s_ssq = sc_stats(style_feats, cnt_s.reshape(_B))

    rb = _pick_block(n)
    nbb = n // rb
    stat_shape = jax.ShapeDtypeStruct((_B, c), jnp.float32)
    stat_spec = pl.BlockSpec((_B, c), lambda i: (0, 0))
    bspec = pl.BlockSpec((1, _B), lambda i: (0, 0))
    c_sum, c_ssq = pl.pallas_call(
        functools.partial(_cstats_kernel, rb, 0),
        grid=(nbb,),
        in_specs=[
            pl.BlockSpec((rb, c), lambda i: (i, 0)),
            bspec,
            bspec,
        ],
        out_specs=[stat_spec, stat_spec],
        out_shape=[stat_shape, stat_shape],
    )(content_feats, st_c, en_c)

    tail = n - x_sc
    rb_t = None
    for cand in (6400, 3200, 1600, 800, 400, 200, 100, 8):
        if tail % cand == 0:
            rb_t = cand
            break
    off_t = x_sc // rb_t
    s_sum_t, s_ssq_t = pl.pallas_call(
        functools.partial(_cstats_kernel, rb_t, x_sc),
        grid=(tail // rb_t,),
        in_specs=[
            pl.BlockSpec((rb_t, c), lambda i: (i + off_t, 0)),
            bspec,
            bspec,
        ],
        out_specs=[stat_spec, stat_spec],
        out_shape=[stat_shape, stat_shape],
    )(style_feats, st_s, en_s)

    w = jnp.asarray(_W)
    scale, offset = pl.pallas_call(
        _finalize_kernel,
        out_shape=[stat_shape, stat_shape],
    )(c_sum, c_ssq, s_sum, s_ssq, s_sum_t, s_ssq_t, cnt_c, cnt_s, w)

    r = _pick_block(n)
    nb = n // r
    bnd_spec = pl.BlockSpec((1, _B), lambda i: (0, 0))
    out = pl.pallas_call(
        functools.partial(_norm_kernel, r),
        grid=(nb,),
        in_specs=[
            pl.BlockSpec((r, c), lambda i: (i, 0)),
            bnd_spec,
            bnd_spec,
            pl.BlockSpec((_B, c), lambda i: (0, 0)),
            pl.BlockSpec((_B, c), lambda i: (0, 0)),
        ],
        out_specs=pl.BlockSpec((r, c), lambda i: (i, 0)),
        out_shape=jax.ShapeDtypeStruct((n, c), jnp.float32),
        compiler_params=pltpu.CompilerParams(
            dimension_semantics=("parallel",)),
    )(content_feats, st_c, en_c, scale, offset)
    return out
